# Initial kernel scaffold; baseline (speedup 1.0000x reference)
import jax
import jax.numpy as jnp
from jax.experimental import pallas as pl


def kernel(x, edge_index, w0, l1_lw, l1_lb, l1_rw, l2_lw, l2_lb, l2_rw, w3):
    raise NotImplementedError("write your pallas kernel here")


# speedup vs baseline: 4.0713x; 4.0713x over previous
import functools

import numpy as np
import jax
import jax.numpy as jnp
from jax import lax
from jax.experimental import pallas as pl
from jax.experimental.pallas import tpu as pltpu
from jax.experimental.pallas import tpu_sc as plsc

B, C, H, W = 2, 128, 112, 112
N = B * H * W
E = 401408
EPS = 1e-4

NSC = 2
NTILE = 16
NH = N // NSC
NTRASH = 256
NHT = NH + NTRASH
K = 128
CHUNKS_PER_TILE = E // K // NTILE
ROWS_OUT_TILE = NH // NTILE
ROWS_ZERO_TILE = NHT // NTILE
INV_SQRT_C = 1.0 / np.sqrt(float(C))


def _sc_agg_body(h_hbm, src_hbm, dst_hbm, out_hbm, cnt_hbm,
                 acc_sh, cnt_sh, rows_v, srcidx_v, dstraw_v, dstrel_v,
                 ones_v, zeros1d_v, sem):
    c = lax.axis_index("c")
    s = lax.axis_index("s")

    zv = jnp.zeros((16,), jnp.float32)

    def zrow(i, carry):
        for l in range(C // 16):
            rows_v[i, pl.ds(l * 16, 16)] = zv
        return carry
    lax.fori_loop(0, K, zrow, 0)

    def z1d(i, carry):
        zeros1d_v[pl.ds(i * 16, 16)] = zv
        return carry
    lax.fori_loop(0, ROWS_ZERO_TILE // 16, z1d, 0)

    ov = jnp.ones((16,), jnp.float32)
    for j in range(K // 16):
        ones_v[pl.ds(j * 16, 16)] = ov

    z0 = s * ROWS_ZERO_TILE
    for off in range(0, ROWS_ZERO_TILE, K):
        nrows = min(K, ROWS_ZERO_TILE - off)
        pltpu.sync_copy(rows_v.at[pl.ds(0, nrows)],
                        acc_sh.at[pl.ds(z0 + off, nrows)])
    pltpu.sync_copy(zeros1d_v, cnt_sh.at[pl.ds(z0, ROWS_ZERO_TILE)])
    plsc.subcore_barrier()

    base = c * NH

    def chunk_body(ct, carry):
        off = ct * K
        pltpu.sync_copy(src_hbm.at[pl.ds(off, K)], srcidx_v)
        pltpu.sync_copy(dst_hbm.at[pl.ds(off, K)], dstraw_v)
        pltpu.async_copy(h_hbm.at[srcidx_v], rows_v, sem).wait()
        for j in range(K // 16):
            d = dstraw_v[pl.ds(j * 16, 16)]
            rel = d - base
            ok = (rel >= 0) & (rel < NH)
            trash = NH + j * 16 + lax.iota(jnp.int32, 16)
            dstrel_v[pl.ds(j * 16, 16)] = jnp.where(ok, rel, trash)
        pltpu.sync_copy(ones_v, cnt_sh.at[dstrel_v], add=True)
        pltpu.sync_copy(rows_v, acc_sh.at[dstrel_v], add=True)
        return carry
    lax.fori_loop(s * CHUNKS_PER_TILE, (s + 1) * CHUNKS_PER_TILE,
                  chunk_body, 0)
    plsc.subcore_barrier()

    abase = s * ROWS_OUT_TILE
    obase = c * NH + s * ROWS_OUT_TILE
    for aoff in range(0, ROWS_OUT_TILE, K):
        nrows = min(K, ROWS_OUT_TILE - aoff)
        pltpu.sync_copy(acc_sh.at[pl.ds(abase + aoff, nrows)],
                        rows_v.at[pl.ds(0, nrows)])
        pltpu.sync_copy(rows_v.at[pl.ds(0, nrows)],
                        out_hbm.at[pl.ds(obase + aoff, nrows)])
    pltpu.sync_copy(cnt_sh.at[pl.ds(abase, ROWS_OUT_TILE)],
                    zeros1d_v.at[pl.ds(0, ROWS_OUT_TILE)])
    pltpu.sync_copy(zeros1d_v.at[pl.ds(0, ROWS_OUT_TILE)],
                    cnt_hbm.at[pl.ds(obase, ROWS_OUT_TILE)])


@functools.cache
def _get_sc_agg():
    mesh = plsc.VectorSubcoreMesh(core_axis_name="c", subcore_axis_name="s",
                                  num_cores=NSC, num_subcores=NTILE)
    return pl.kernel(
        _sc_agg_body,
        out_type=[jax.ShapeDtypeStruct((N, C), jnp.float32),
                  jax.ShapeDtypeStruct((N,), jnp.float32)],
        mesh=mesh,
        scratch_types=[
            pltpu.VMEM_SHARED((NHT, C), jnp.float32),
            pltpu.VMEM_SHARED((NHT,), jnp.float32),
            pltpu.VMEM((K, C), jnp.float32),
            pltpu.VMEM((K,), jnp.int32),
            pltpu.VMEM((K,), jnp.int32),
            pltpu.VMEM((K,), jnp.int32),
            pltpu.VMEM((K,), jnp.float32),
            pltpu.VMEM((ROWS_ZERO_TILE,), jnp.float32),
            pltpu.SemaphoreType.DMA,
        ],
    )


_R = 3584
_G = N // _R


def _normw(w):
    n = jnp.sqrt(jnp.sum(w * w, axis=1, keepdims=True))
    return w / (EPS + n * INV_SQRT_C)


def _mm_nt(a, w):
    return lax.dot_general(a, w, (((1,), (1,)), ((), ())),
                           preferred_element_type=jnp.float32)


def _mpconv_body(x_ref, w_ref, o_ref):
    wn = _normw(w_ref[...]) * INV_SQRT_C
    o_ref[...] = _mm_nt(x_ref[...], wn)


def _sage_dense(h, sums, cnt, lw, lb, rw):
    agg = sums * (1.0 / jnp.maximum(cnt, 1.0))
    lwn = _normw(lw) * INV_SQRT_C
    rwn = _normw(rw) * INV_SQRT_C
    out = _mm_nt(agg, lwn) + lb + _mm_nt(h, rwn)
    nrm = jnp.sqrt(jnp.sum(out * out, axis=1, keepdims=True))
    return out / jnp.maximum(nrm, 1e-12)


def _sage_body(h_ref, sums_ref, cnt_ref, lw_ref, lb_ref, rw_ref, o_ref):
    o_ref[...] = _sage_dense(h_ref[...], sums_ref[...], cnt_ref[...],
                             lw_ref[...], lb_ref[...], rw_ref[...])


def _sage_final_body(h_ref, sums_ref, cnt_ref, lw_ref, lb_ref, rw_ref,
                     w3_ref, o_ref):
    hn = _sage_dense(h_ref[...], sums_ref[...], cnt_ref[...], lw_ref[...],
                     lb_ref[...], rw_ref[...])
    w3n = _normw(w3_ref[...]) * INV_SQRT_C
    o_ref[...] = _mm_nt(hn, w3n)


_row_spec = pl.BlockSpec((_R, C), lambda i: (i, 0))
_cnt_spec = pl.BlockSpec((_R, 1), lambda i: (i, 0))
_w_spec = pl.BlockSpec((C, C), lambda i: (0, 0))
_b_spec = pl.BlockSpec((1, C), lambda i: (0, 0))
_out_sds = jax.ShapeDtypeStruct((N, C), jnp.float32)

_mpconv = pl.pallas_call(
    _mpconv_body, grid=(_G,),
    in_specs=[_row_spec, _w_spec],
    out_specs=_row_spec, out_shape=_out_sds)

_sage_tc = pl.pallas_call(
    _sage_body, grid=(_G,),
    in_specs=[_row_spec, _row_spec, _cnt_spec, _w_spec, _b_spec, _w_spec],
    out_specs=_row_spec, out_shape=_out_sds)

_sage_final_tc = pl.pallas_call(
    _sage_final_body, grid=(_G,),
    in_specs=[_row_spec, _row_spec, _cnt_spec, _w_spec, _b_spec, _w_spec,
              _w_spec],
    out_specs=_row_spec, out_shape=_out_sds)


def kernel(x, edge_index, w0, l1_lw, l1_lb, l1_rw, l2_lw, l2_lb, l2_rw, w3):
    nodes = jnp.transpose(x, (0, 2, 3, 1)).reshape(-1, C)
    src = edge_index[0]
    dst = edge_index[1]
    sc_agg = _get_sc_agg()
    h0 = _mpconv(nodes, w0)
    sums1, cnt1 = sc_agg(h0, src, dst)
    h1 = _sage_tc(h0, sums1, cnt1.reshape(N, 1), l1_lw,
                  l1_lb.reshape(1, C), l1_rw)
    sums2, cnt2 = sc_agg(h1, src, dst)
    y = _sage_final_tc(h1, sums2, cnt2.reshape(N, 1), l2_lw,
                       l2_lb.reshape(1, C), l2_rw, w3)
    return jnp.transpose(y.reshape(B, H, W, C), (0, 3, 1, 2))


# speedup vs baseline: 6.2116x; 1.5257x over previous
import functools

import numpy as np
import jax
import jax.numpy as jnp
from jax import lax
from jax.experimental import pallas as pl
from jax.experimental.pallas import tpu as pltpu
from jax.experimental.pallas import tpu_sc as plsc

B, C, H, W = 2, 128, 112, 112
N = B * H * W
E = 401408
EPS = 1e-4

NSC = 2
NTILE = 16
NH = N // NSC
NTRASH = 256
NHT = NH + NTRASH
K = 64
IB = 8
KB = K * IB
EDGES_PER_TILE = E // NTILE
BATCHES_PER_TILE = EDGES_PER_TILE // KB
ROWS_OUT_TILE = NH // NTILE
ROWS_ZERO_TILE = NHT // NTILE
INV_SQRT_C = 1.0 / np.sqrt(float(C))


def _sc_agg_impl(h_hbm, src_hbm, dst_hbm, out_hbm, cnt_hbm,
                 acc_sh, cnt_sh, rowsA, rowsB, srcidx_v, dstraw_v, dstrel_v,
                 ones_v, zeros1d_v, semA, semB):
    with_cnt = cnt_hbm is not None
    c = lax.axis_index("c")
    s = lax.axis_index("s")

    zv = jnp.zeros((16,), jnp.float32)

    def zrow(i, carry):
        for l in range(C // 16):
            rowsA[i, pl.ds(l * 16, 16)] = zv
        return carry
    lax.fori_loop(0, K, zrow, 0)

    def z1d(i, carry):
        zeros1d_v[pl.ds(i * 16, 16)] = zv
        return carry
    lax.fori_loop(0, ROWS_ZERO_TILE // 16, z1d, 0)

    if with_cnt:
        ov = jnp.ones((16,), jnp.float32)
        for j in range(K // 16):
            ones_v[pl.ds(j * 16, 16)] = ov

    z0 = s * ROWS_ZERO_TILE
    for off in range(0, ROWS_ZERO_TILE, K):
        nrows = min(K, ROWS_ZERO_TILE - off)
        pltpu.sync_copy(rowsA.at[pl.ds(0, nrows)],
                        acc_sh.at[pl.ds(z0 + off, nrows)])
    if with_cnt:
        pltpu.sync_copy(zeros1d_v, cnt_sh.at[pl.ds(z0, ROWS_ZERO_TILE)])
    plsc.subcore_barrier()

    base = c * NH

    def batch_body(b, carry):
        eoff = s * EDGES_PER_TILE + b * KB
        pltpu.sync_copy(src_hbm.at[pl.ds(eoff, KB)], srcidx_v)
        pltpu.sync_copy(dst_hbm.at[pl.ds(eoff, KB)], dstraw_v)
        for j in range(KB // 16):
            d = dstraw_v[pl.ds(j * 16, 16)]
            rel = d - base
            ok = (rel >= 0) & (rel < NH)
            trash = NH + (j % 16) * 16 + lax.iota(jnp.int32, 16)
            dstrel_v[j // (K // 16), pl.ds((j % (K // 16)) * 16, 16)] = (
                jnp.where(ok, rel, trash))

        def start_gather(k, buf, sem):
            return pltpu.async_copy(
                h_hbm.at[srcidx_v.at[pl.ds(k * K, K)]], buf, sem)

        g = start_gather(0, rowsA, semA)
        for k in range(IB):
            cur_buf = rowsA if k % 2 == 0 else rowsB
            nxt_buf = rowsB if k % 2 == 0 else rowsA
            nxt_sem = semB if k % 2 == 0 else semA
            g_next = start_gather(k + 1, nxt_buf, nxt_sem) if k + 1 < IB else None
            g.wait()
            if with_cnt:
                pltpu.sync_copy(ones_v, cnt_sh.at[dstrel_v.at[k]], add=True)
            pltpu.sync_copy(cur_buf, acc_sh.at[dstrel_v.at[k]], add=True)
            g = g_next
        return carry
    lax.fori_loop(0, BATCHES_PER_TILE, batch_body, 0)
    plsc.subcore_barrier()

    abase = s * ROWS_OUT_TILE
    obase = c * NH + s * ROWS_OUT_TILE
    for aoff in range(0, ROWS_OUT_TILE, K):
        nrows = min(K, ROWS_OUT_TILE - aoff)
        pltpu.sync_copy(acc_sh.at[pl.ds(abase + aoff, nrows)],
                        rowsA.at[pl.ds(0, nrows)])
        pltpu.sync_copy(rowsA.at[pl.ds(0, nrows)],
                        out_hbm.at[pl.ds(obase + aoff, nrows)])
    if with_cnt:
        pltpu.sync_copy(cnt_sh.at[pl.ds(abase, ROWS_OUT_TILE)],
                        zeros1d_v.at[pl.ds(0, ROWS_OUT_TILE)])
        pltpu.sync_copy(zeros1d_v.at[pl.ds(0, ROWS_OUT_TILE)],
                        cnt_hbm.at[pl.ds(obase, ROWS_OUT_TILE)])


def _sc_body_cnt(h_hbm, src_hbm, dst_hbm, out_hbm, cnt_hbm, acc_sh, cnt_sh,
                 rowsA, rowsB, srcidx_v, dstraw_v, dstrel_v, ones_v,
                 zeros1d_v, semA, semB):
    _sc_agg_impl(h_hbm, src_hbm, dst_hbm, out_hbm, cnt_hbm, acc_sh, cnt_sh,
                 rowsA, rowsB, srcidx_v, dstraw_v, dstrel_v, ones_v,
                 zeros1d_v, semA, semB)


def _sc_body_nocnt(h_hbm, src_hbm, dst_hbm, out_hbm, acc_sh,
                   rowsA, rowsB, srcidx_v, dstraw_v, dstrel_v,
                   zeros1d_v, semA, semB):
    _sc_agg_impl(h_hbm, src_hbm, dst_hbm, out_hbm, None, acc_sh, None,
                 rowsA, rowsB, srcidx_v, dstraw_v, dstrel_v, None,
                 zeros1d_v, semA, semB)


def _sc_common_scratch():
    return [
        pltpu.VMEM((K, C), jnp.float32),
        pltpu.VMEM((K, C), jnp.float32),
        pltpu.VMEM((KB,), jnp.int32),
        pltpu.VMEM((KB,), jnp.int32),
        pltpu.VMEM((IB, K), jnp.int32),
    ]


@functools.cache
def _get_sc_aggs():
    mesh = plsc.VectorSubcoreMesh(core_axis_name="c", subcore_axis_name="s",
                                  num_cores=NSC, num_subcores=NTILE)
    agg_cnt = pl.kernel(
        _sc_body_cnt,
        out_type=[jax.ShapeDtypeStruct((N, C), jnp.float32),
                  jax.ShapeDtypeStruct((N,), jnp.float32)],
        mesh=mesh,
        scratch_types=[
            pltpu.VMEM_SHARED((NHT, C), jnp.float32),
            pltpu.VMEM_SHARED((NHT,), jnp.float32),
            *_sc_common_scratch(),
            pltpu.VMEM((K,), jnp.float32),
            pltpu.VMEM((ROWS_ZERO_TILE,), jnp.float32),
            pltpu.SemaphoreType.DMA,
            pltpu.SemaphoreType.DMA,
        ],
    )
    agg_nocnt = pl.kernel(
        _sc_body_nocnt,
        out_type=jax.ShapeDtypeStruct((N, C), jnp.float32),
        mesh=mesh,
        scratch_types=[
            pltpu.VMEM_SHARED((NHT, C), jnp.float32),
            *_sc_common_scratch(),
            pltpu.VMEM((ROWS_ZERO_TILE,), jnp.float32),
            pltpu.SemaphoreType.DMA,
            pltpu.SemaphoreType.DMA,
        ],
    )
    return agg_cnt, agg_nocnt


_R = 3584
_G = N // _R


def _normw(w):
    n = jnp.sqrt(jnp.sum(w * w, axis=1, keepdims=True))
    return w / (EPS + n * INV_SQRT_C)


def _mm_nt(a, w):
    return lax.dot_general(a, w, (((1,), (1,)), ((), ())),
                           preferred_element_type=jnp.float32)


def _mpconv_body(x_ref, w_ref, o_ref):
    wn = _normw(w_ref[...]) * INV_SQRT_C
    o_ref[...] = _mm_nt(x_ref[...], wn)


def _sage_dense(h, sums, cnt, lw, lb, rw):
    agg = sums * (1.0 / jnp.maximum(cnt, 1.0))
    lwn = _normw(lw) * INV_SQRT_C
    rwn = _normw(rw) * INV_SQRT_C
    out = _mm_nt(agg, lwn) + lb + _mm_nt(h, rwn)
    nrm = jnp.sqrt(jnp.sum(out * out, axis=1, keepdims=True))
    return out / jnp.maximum(nrm, 1e-12)


def _sage_body(h_ref, sums_ref, cnt_ref, lw_ref, lb_ref, rw_ref, o_ref):
    o_ref[...] = _sage_dense(h_ref[...], sums_ref[...], cnt_ref[...],
                             lw_ref[...], lb_ref[...], rw_ref[...])


def _sage_final_body(h_ref, sums_ref, cnt_ref, lw_ref, lb_ref, rw_ref,
                     w3_ref, o_ref):
    hn = _sage_dense(h_ref[...], sums_ref[...], cnt_ref[...], lw_ref[...],
                     lb_ref[...], rw_ref[...])
    w3n = _normw(w3_ref[...]) * INV_SQRT_C
    o_ref[...] = _mm_nt(hn, w3n)


_row_spec = pl.BlockSpec((_R, C), lambda i: (i, 0))
_cnt_spec = pl.BlockSpec((_R, 1), lambda i: (i, 0))
_w_spec = pl.BlockSpec((C, C), lambda i: (0, 0))
_b_spec = pl.BlockSpec((1, C), lambda i: (0, 0))
_out_sds = jax.ShapeDtypeStruct((N, C), jnp.float32)

_mpconv = pl.pallas_call(
    _mpconv_body, grid=(_G,),
    in_specs=[_row_spec, _w_spec],
    out_specs=_row_spec, out_shape=_out_sds)

_sage_tc = pl.pallas_call(
    _sage_body, grid=(_G,),
    in_specs=[_row_spec, _row_spec, _cnt_spec, _w_spec, _b_spec, _w_spec],
    out_specs=_row_spec, out_shape=_out_sds)

_sage_final_tc = pl.pallas_call(
    _sage_final_body, grid=(_G,),
    in_specs=[_row_spec, _row_spec, _cnt_spec, _w_spec, _b_spec, _w_spec,
              _w_spec],
    out_specs=_row_spec, out_shape=_out_sds)


def kernel(x, edge_index, w0, l1_lw, l1_lb, l1_rw, l2_lw, l2_lb, l2_rw, w3):
    nodes = jnp.transpose(x, (0, 2, 3, 1)).reshape(-1, C)
    src = edge_index[0]
    dst = edge_index[1]
    agg_cnt, agg_nocnt = _get_sc_aggs()
    h0 = _mpconv(nodes, w0)
    sums1, cnt1 = agg_cnt(h0, src, dst)
    cnt1c = cnt1.reshape(N, 1)
    h1 = _sage_tc(h0, sums1, cnt1c, l1_lw, l1_lb.reshape(1, C), l1_rw)
    sums2 = agg_nocnt(h1, src, dst)
    y = _sage_final_tc(h1, sums2, cnt1c, l2_lw, l2_lb.reshape(1, C), l2_rw, w3)
    return jnp.transpose(y.reshape(B, H, W, C), (0, 3, 1, 2))


# speedup vs baseline: 7.2083x; 1.1605x over previous
import functools

import numpy as np
import jax
import jax.numpy as jnp
from jax import lax
from jax.experimental import pallas as pl
from jax.experimental.pallas import tpu as pltpu
from jax.experimental.pallas import tpu_sc as plsc

B, C, H, W = 2, 128, 112, 112
N = B * H * W
E = 401408
EPS = 1e-4

NSC = 2
NTILE = 16
NW = NSC * NTILE
NH = N // NSC
NTRASH = 256
NHT = NH + NTRASH
K = 64
IB = 8
KB = K * IB
PK = KB
EDGES_PER_TILE = E // NTILE
PART_KB = 896
PART_BATCHES = EDGES_PER_TILE // PART_KB
RCAP = EDGES_PER_TILE
ROWS_OUT_TILE = NH // NTILE
ROWS_ZERO_TILE = NHT // NTILE
INV_SQRT_C = 1.0 / np.sqrt(float(C))


def _part_body(src_hbm, dst_hbm, srcp_hbm, dstp_hbm, cnts_hbm,
               srcraw_v, dstraw_v, pkstg_v, srcstg_v, dststg_v, cntbuf_v):
    c = lax.axis_index("c")
    s = lax.axis_index("s")
    r = c * NTILE + s
    base = c * NH
    rbase = r * RCAP
    iota = lax.iota(jnp.int32, 16)

    def flush(p, nf):
        cond = p >= PK

        @pl.when(cond)
        def _():
            for q in range(PK // 16):
                v = pkstg_v[pl.ds(q * 16, 16)]
                srcstg_v[pl.ds(q * 16, 16)] = v >> 14
                dststg_v[pl.ds(q * 16, 16)] = v & 16383
            pltpu.sync_copy(srcstg_v, srcp_hbm.at[pl.ds(rbase + nf * PK, PK)])
            pltpu.sync_copy(dststg_v, dstp_hbm.at[pl.ds(rbase + nf * PK, PK)])
            v = pkstg_v[pl.ds(PK, 16)]
            pkstg_v[pl.ds(0, 16)] = v
        p2 = jnp.where(cond, p - PK, p)
        nf2 = jnp.where(cond, nf + 1, nf)
        return p2, nf2

    def batch_body(b, carry):
        p, f = carry
        eoff = s * EDGES_PER_TILE + b * PART_KB
        pltpu.sync_copy(src_hbm.at[pl.ds(eoff, PART_KB)], srcraw_v)
        pltpu.sync_copy(dst_hbm.at[pl.ds(eoff, PART_KB)], dstraw_v)
        for j in range(PART_KB // 16):
            sr = srcraw_v[pl.ds(j * 16, 16)]
            dr = dstraw_v[pl.ds(j * 16, 16)]
            rel = dr - base
            ok = (rel >= 0) & (rel < NH)
            key = jnp.where(ok, 0, 1).astype(jnp.int32)
            packed = (sr << 14) | (rel & 16383)
            _, sval = plsc.sort_key_val(key, packed)
            cnt16 = plsc.all_reduce_population_count(ok)[0]
            plsc.store_scatter(pkstg_v, [p + iota], sval,
                               mask=iota < cnt16)
            p = p + cnt16
            p, f = flush(p, f)
        return p, f

    p, f = lax.fori_loop(0, PART_BATCHES, batch_body,
                         (jnp.int32(0), jnp.int32(0)))

    need16 = (16 - (p % 16)) % 16
    mask = iota < need16
    plsc.store_scatter(pkstg_v, [p + iota], (iota << 14) | (NH + iota),
                       mask=mask)
    p = p + need16
    p, f = flush(p, f)

    total = p + f * PK
    n_push = jnp.where(total == 0, PK // 16, ((PK - (p % PK)) % PK) // 16)

    def push_body(j, carry):
        p, f = carry
        plsc.store_scatter(pkstg_v, [p + iota],
                           (iota << 14) | (NH + (j % 16) * 16 + iota))
        p = p + 16
        p, f = flush(p, f)
        return p, f
    p, f = lax.fori_loop(0, n_push, push_body, (p, f))

    cntbuf_v[...] = jnp.zeros((16,), jnp.int32) + f
    pltpu.sync_copy(cntbuf_v, cnts_hbm.at[pl.ds(r * 16, 16)])


@functools.cache
def _get_partition():
    mesh = plsc.VectorSubcoreMesh(core_axis_name="c", subcore_axis_name="s",
                                  num_cores=NSC, num_subcores=NTILE)
    return pl.kernel(
        _part_body,
        out_type=[jax.ShapeDtypeStruct((NW * RCAP,), jnp.int32),
                  jax.ShapeDtypeStruct((NW * RCAP,), jnp.int32),
                  jax.ShapeDtypeStruct((NW * 16,), jnp.int32)],
        mesh=mesh,
        compiler_params=pltpu.CompilerParams(needs_layout_passes=False),
        scratch_types=[
            pltpu.VMEM((PART_KB,), jnp.int32),
            pltpu.VMEM((PART_KB,), jnp.int32),
            pltpu.VMEM((PK + 16,), jnp.int32),
            pltpu.VMEM((PK,), jnp.int32),
            pltpu.VMEM((PK,), jnp.int32),
            pltpu.VMEM((16,), jnp.int32),
        ],
    )


def _sc_agg_impl(h_hbm, srcp_hbm, dstp_hbm, cnts_hbm, out_hbm, cnt_hbm,
                 acc_sh, cnt_sh, rowsA, rowsB, srcidx_v, dstraw_v, dstrel_v,
                 ones_v, zeros1d_v, cntbuf_v, semA, semB):
    with_cnt = cnt_hbm is not None
    c = lax.axis_index("c")
    s = lax.axis_index("s")
    r = c * NTILE + s
    rbase = r * RCAP

    pltpu.sync_copy(cnts_hbm.at[pl.ds(r * 16, 16)], cntbuf_v)
    nb = cntbuf_v[...][0]

    zv = jnp.zeros((16,), jnp.float32)

    def zrow(i, carry):
        for l in range(C // 16):
            rowsA[i, pl.ds(l * 16, 16)] = zv
        return carry
    lax.fori_loop(0, K, zrow, 0)

    def z1d(i, carry):
        zeros1d_v[pl.ds(i * 16, 16)] = zv
        return carry
    lax.fori_loop(0, ROWS_ZERO_TILE // 16, z1d, 0)

    if with_cnt:
        ov = jnp.ones((16,), jnp.float32)
        for j in range(K // 16):
            ones_v[pl.ds(j * 16, 16)] = ov

    z0 = s * ROWS_ZERO_TILE
    for off in range(0, ROWS_ZERO_TILE, K):
        nrows = min(K, ROWS_ZERO_TILE - off)
        pltpu.sync_copy(rowsA.at[pl.ds(0, nrows)],
                        acc_sh.at[pl.ds(z0 + off, nrows)])
    if with_cnt:
        pltpu.sync_copy(zeros1d_v, cnt_sh.at[pl.ds(z0, ROWS_ZERO_TILE)])
    plsc.subcore_barrier()

    def batch_body(b, carry):
        eoff = rbase + b * KB
        pltpu.sync_copy(srcp_hbm.at[pl.ds(eoff, KB)], srcidx_v)
        pltpu.sync_copy(dstp_hbm.at[pl.ds(eoff, KB)], dstraw_v)
        for j in range(KB // 16):
            dstrel_v[j // (K // 16), pl.ds((j % (K // 16)) * 16, 16)] = (
                dstraw_v[pl.ds(j * 16, 16)])

        def start_gather(k, buf, sem):
            return pltpu.async_copy(
                h_hbm.at[srcidx_v.at[pl.ds(k * K, K)]], buf, sem)

        g = start_gather(0, rowsA, semA)
        for k in range(IB):
            cur_buf = rowsA if k % 2 == 0 else rowsB
            nxt_buf = rowsB if k % 2 == 0 else rowsA
            nxt_sem = semB if k % 2 == 0 else semA
            g_next = start_gather(k + 1, nxt_buf, nxt_sem) if k + 1 < IB else None
            g.wait()
            if with_cnt:
                pltpu.sync_copy(ones_v, cnt_sh.at[dstrel_v.at[k]], add=True)
            pltpu.sync_copy(cur_buf, acc_sh.at[dstrel_v.at[k]], add=True)
            g = g_next
        return carry
    lax.fori_loop(0, nb, batch_body, 0)
    plsc.subcore_barrier()

    abase = s * ROWS_OUT_TILE
    obase = c * NH + s * ROWS_OUT_TILE
    for aoff in range(0, ROWS_OUT_TILE, K):
        nrows = min(K, ROWS_OUT_TILE - aoff)
        pltpu.sync_copy(acc_sh.at[pl.ds(abase + aoff, nrows)],
                        rowsA.at[pl.ds(0, nrows)])
        pltpu.sync_copy(rowsA.at[pl.ds(0, nrows)],
                        out_hbm.at[pl.ds(obase + aoff, nrows)])
    if with_cnt:
        pltpu.sync_copy(cnt_sh.at[pl.ds(abase, ROWS_OUT_TILE)],
                        zeros1d_v.at[pl.ds(0, ROWS_OUT_TILE)])
        pltpu.sync_copy(zeros1d_v.at[pl.ds(0, ROWS_OUT_TILE)],
                        cnt_hbm.at[pl.ds(obase, ROWS_OUT_TILE)])


def _sc_body_cnt(h_hbm, srcp_hbm, dstp_hbm, cnts_hbm, out_hbm, cnt_hbm,
                 acc_sh, cnt_sh, rowsA, rowsB, srcidx_v, dstraw_v, dstrel_v,
                 ones_v, zeros1d_v, cntbuf_v, semA, semB):
    _sc_agg_impl(h_hbm, srcp_hbm, dstp_hbm, cnts_hbm, out_hbm, cnt_hbm,
                 acc_sh, cnt_sh, rowsA, rowsB, srcidx_v, dstraw_v, dstrel_v,
                 ones_v, zeros1d_v, cntbuf_v, semA, semB)


def _sc_body_nocnt(h_hbm, srcp_hbm, dstp_hbm, cnts_hbm, out_hbm, acc_sh,
                   rowsA, rowsB, srcidx_v, dstraw_v, dstrel_v,
                   zeros1d_v, cntbuf_v, semA, semB):
    _sc_agg_impl(h_hbm, srcp_hbm, dstp_hbm, cnts_hbm, out_hbm, None,
                 acc_sh, None, rowsA, rowsB, srcidx_v, dstraw_v, dstrel_v,
                 None, zeros1d_v, cntbuf_v, semA, semB)


def _sc_common_scratch():
    return [
        pltpu.VMEM((K, C), jnp.float32),
        pltpu.VMEM((K, C), jnp.float32),
        pltpu.VMEM((KB,), jnp.int32),
        pltpu.VMEM((KB,), jnp.int32),
        pltpu.VMEM((IB, K), jnp.int32),
    ]


@functools.cache
def _get_sc_aggs():
    mesh = plsc.VectorSubcoreMesh(core_axis_name="c", subcore_axis_name="s",
                                  num_cores=NSC, num_subcores=NTILE)
    agg_cnt = pl.kernel(
        _sc_body_cnt,
        out_type=[jax.ShapeDtypeStruct((N, C), jnp.float32),
                  jax.ShapeDtypeStruct((N,), jnp.float32)],
        mesh=mesh,
        scratch_types=[
            pltpu.VMEM_SHARED((NHT, C), jnp.float32),
            pltpu.VMEM_SHARED((NHT,), jnp.float32),
            *_sc_common_scratch(),
            pltpu.VMEM((K,), jnp.float32),
            pltpu.VMEM((ROWS_ZERO_TILE,), jnp.float32),
            pltpu.VMEM((16,), jnp.int32),
            pltpu.SemaphoreType.DMA,
            pltpu.SemaphoreType.DMA,
        ],
    )
    agg_nocnt = pl.kernel(
        _sc_body_nocnt,
        out_type=jax.ShapeDtypeStruct((N, C), jnp.float32),
        mesh=mesh,
        scratch_types=[
            pltpu.VMEM_SHARED((NHT, C), jnp.float32),
            *_sc_common_scratch(),
            pltpu.VMEM((ROWS_ZERO_TILE,), jnp.float32),
            pltpu.VMEM((16,), jnp.int32),
            pltpu.SemaphoreType.DMA,
            pltpu.SemaphoreType.DMA,
        ],
    )
    return agg_cnt, agg_nocnt


_R = 3584
_G = N // _R


def _normw(w):
    n = jnp.sqrt(jnp.sum(w * w, axis=1, keepdims=True))
    return w / (EPS + n * INV_SQRT_C)


def _mm_nt(a, w):
    return lax.dot_general(a, w, (((1,), (1,)), ((), ())),
                           preferred_element_type=jnp.float32)


def _mpconv_body(x_ref, w_ref, o_ref):
    wn = _normw(w_ref[...]) * INV_SQRT_C
    o_ref[...] = _mm_nt(x_ref[...], wn)


def _sage_dense(h, sums, cnt, lw, lb, rw):
    agg = sums * (1.0 / jnp.maximum(cnt, 1.0))
    lwn = _normw(lw) * INV_SQRT_C
    rwn = _normw(rw) * INV_SQRT_C
    out = _mm_nt(agg, lwn) + lb + _mm_nt(h, rwn)
    nrm = jnp.sqrt(jnp.sum(out * out, axis=1, keepdims=True))
    return out / jnp.maximum(nrm, 1e-12)


def _sage_body(h_ref, sums_ref, cnt_ref, lw_ref, lb_ref, rw_ref, o_ref):
    o_ref[...] = _sage_dense(h_ref[...], sums_ref[...], cnt_ref[...],
                             lw_ref[...], lb_ref[...], rw_ref[...])


def _sage_final_body(h_ref, sums_ref, cnt_ref, lw_ref, lb_ref, rw_ref,
                     w3_ref, o_ref):
    hn = _sage_dense(h_ref[...], sums_ref[...], cnt_ref[...], lw_ref[...],
                     lb_ref[...], rw_ref[...])
    w3n = _normw(w3_ref[...]) * INV_SQRT_C
    o_ref[...] = _mm_nt(hn, w3n)


_row_spec = pl.BlockSpec((_R, C), lambda i: (i, 0))
_cnt_spec = pl.BlockSpec((_R, 1), lambda i: (i, 0))
_w_spec = pl.BlockSpec((C, C), lambda i: (0, 0))
_b_spec = pl.BlockSpec((1, C), lambda i: (0, 0))
_out_sds = jax.ShapeDtypeStruct((N, C), jnp.float32)

_mpconv = pl.pallas_call(
    _mpconv_body, grid=(_G,),
    in_specs=[_row_spec, _w_spec],
    out_specs=_row_spec, out_shape=_out_sds)

_sage_tc = pl.pallas_call(
    _sage_body, grid=(_G,),
    in_specs=[_row_spec, _row_spec, _cnt_spec, _w_spec, _b_spec, _w_spec],
    out_specs=_row_spec, out_shape=_out_sds)

_sage_final_tc = pl.pallas_call(
    _sage_final_body, grid=(_G,),
    in_specs=[_row_spec, _row_spec, _cnt_spec, _w_spec, _b_spec, _w_spec,
              _w_spec],
    out_specs=_row_spec, out_shape=_out_sds)


def kernel(x, edge_index, w0, l1_lw, l1_lb, l1_rw, l2_lw, l2_lb, l2_rw, w3):
    nodes = jnp.transpose(x, (0, 2, 3, 1)).reshape(-1, C)
    src = edge_index[0]
    dst = edge_index[1]
    partition = _get_partition()
    agg_cnt, agg_nocnt = _get_sc_aggs()
    srcp, dstp, cnts = partition(src, dst)
    h0 = _mpconv(nodes, w0)
    sums1, cnt1 = agg_cnt(h0, srcp, dstp, cnts)
    cnt1c = cnt1.reshape(N, 1)
    h1 = _sage_tc(h0, sums1, cnt1c, l1_lw, l1_lb.reshape(1, C), l1_rw)
    sums2 = agg_nocnt(h1, srcp, dstp, cnts)
    y = _sage_final_tc(h1, sums2, cnt1c, l2_lw, l2_lb.reshape(1, C), l2_rw, w3)
    return jnp.transpose(y.reshape(B, H, W, C), (0, 3, 1, 2))


# speedup vs baseline: 7.2640x; 1.0077x over previous
import functools

import numpy as np
import jax
import jax.numpy as jnp
from jax import lax
from jax.experimental import pallas as pl
from jax.experimental.pallas import tpu as pltpu
from jax.experimental.pallas import tpu_sc as plsc

B, C, H, W = 2, 128, 112, 112
N = B * H * W
E = 401408
EPS = 1e-4

NSC = 2
NTILE = 16
NW = NSC * NTILE
NH = N // NSC
NTRASH = 256
NHT = NH + NTRASH
K = 64
IB = 8
KB = K * IB
PK = KB
EDGES_PER_TILE = E // NTILE
PART_KB = 896
PART_BATCHES = EDGES_PER_TILE // PART_KB
RCAP = EDGES_PER_TILE
ROWS_OUT_TILE = NH // NTILE
ROWS_ZERO_TILE = NHT // NTILE
INV_SQRT_C = 1.0 / np.sqrt(float(C))


def _part_body(src_hbm, dst_hbm, srcp_hbm, dstp_hbm, cnts_hbm,
               srcraw_v, dstraw_v, pkstg_v, srcstg_v, dststg_v, cntbuf_v,
               semS, semD):
    c = lax.axis_index("c")
    s = lax.axis_index("s")
    r = c * NTILE + s
    base = c * NH
    rbase = r * RCAP
    iota = lax.iota(jnp.int32, 16)

    def flush(p, nf):
        cond = p >= PK

        @pl.when(cond)
        def _():
            @pl.when(nf > 0)
            def _():
                pltpu.make_async_copy(
                    srcstg_v,
                    srcp_hbm.at[pl.ds(rbase + (nf - 1) * PK, PK)],
                    semS).wait()
                pltpu.make_async_copy(
                    dststg_v,
                    dstp_hbm.at[pl.ds(rbase + (nf - 1) * PK, PK)],
                    semD).wait()
            for q in range(PK // 16):
                v = pkstg_v[pl.ds(q * 16, 16)]
                srcstg_v[pl.ds(q * 16, 16)] = v >> 14
                dststg_v[pl.ds(q * 16, 16)] = v & 16383
            pltpu.async_copy(srcstg_v,
                             srcp_hbm.at[pl.ds(rbase + nf * PK, PK)], semS)
            pltpu.async_copy(dststg_v,
                             dstp_hbm.at[pl.ds(rbase + nf * PK, PK)], semD)
            v = pkstg_v[pl.ds(PK, 16)]
            pkstg_v[pl.ds(0, 16)] = v
        p2 = jnp.where(cond, p - PK, p)
        nf2 = jnp.where(cond, nf + 1, nf)
        return p2, nf2

    def batch_body(b, carry):
        p, f = carry
        eoff = s * EDGES_PER_TILE + b * PART_KB
        pltpu.sync_copy(src_hbm.at[pl.ds(eoff, PART_KB)], srcraw_v)
        pltpu.sync_copy(dst_hbm.at[pl.ds(eoff, PART_KB)], dstraw_v)
        for j in range(PART_KB // 16):
            sr = srcraw_v[pl.ds(j * 16, 16)]
            dr = dstraw_v[pl.ds(j * 16, 16)]
            rel = dr - base
            ok = (rel >= 0) & (rel < NH)
            key = jnp.where(ok, 0, 1).astype(jnp.int32)
            packed = (sr << 14) | (rel & 16383)
            _, sval = plsc.sort_key_val(key, packed)
            cnt16 = plsc.all_reduce_population_count(ok)[0]
            plsc.store_scatter(pkstg_v, [p + iota], sval,
                               mask=iota < cnt16)
            p = p + cnt16
            p, f = flush(p, f)
        return p, f

    p, f = lax.fori_loop(0, PART_BATCHES, batch_body,
                         (jnp.int32(0), jnp.int32(0)))

    need16 = (16 - (p % 16)) % 16
    mask = iota < need16
    plsc.store_scatter(pkstg_v, [p + iota], (iota << 14) | (NH + iota),
                       mask=mask)
    p = p + need16
    p, f = flush(p, f)

    total = p + f * PK
    n_push = jnp.where(total == 0, PK // 16, ((PK - (p % PK)) % PK) // 16)

    def push_body(j, carry):
        p, f = carry
        plsc.store_scatter(pkstg_v, [p + iota],
                           (iota << 14) | (NH + (j % 16) * 16 + iota))
        p = p + 16
        p, f = flush(p, f)
        return p, f
    p, f = lax.fori_loop(0, n_push, push_body, (p, f))

    @pl.when(f > 0)
    def _():
        pltpu.make_async_copy(
            srcstg_v, srcp_hbm.at[pl.ds(rbase + (f - 1) * PK, PK)],
            semS).wait()
        pltpu.make_async_copy(
            dststg_v, dstp_hbm.at[pl.ds(rbase + (f - 1) * PK, PK)],
            semD).wait()
    cntbuf_v[...] = jnp.zeros((16,), jnp.int32) + f
    pltpu.sync_copy(cntbuf_v, cnts_hbm.at[pl.ds(r * 16, 16)])


@functools.cache
def _get_partition():
    mesh = plsc.VectorSubcoreMesh(core_axis_name="c", subcore_axis_name="s",
                                  num_cores=NSC, num_subcores=NTILE)
    return pl.kernel(
        _part_body,
        out_type=[jax.ShapeDtypeStruct((NW * RCAP,), jnp.int32),
                  jax.ShapeDtypeStruct((NW * RCAP,), jnp.int32),
                  jax.ShapeDtypeStruct((NW * 16,), jnp.int32)],
        mesh=mesh,
        compiler_params=pltpu.CompilerParams(needs_layout_passes=False),
        scratch_types=[
            pltpu.VMEM((PART_KB,), jnp.int32),
            pltpu.VMEM((PART_KB,), jnp.int32),
            pltpu.VMEM((PK + 16,), jnp.int32),
            pltpu.VMEM((PK,), jnp.int32),
            pltpu.VMEM((PK,), jnp.int32),
            pltpu.VMEM((16,), jnp.int32),
            pltpu.SemaphoreType.DMA,
            pltpu.SemaphoreType.DMA,
        ],
    )


def _sc_agg_impl(h_hbm, srcp_hbm, dstp_hbm, cnts_hbm, out_hbm, cnt_hbm,
                 acc_sh, cnt_sh, rowsA, rowsB, srcidx_v, dstraw_v, dstrel_v,
                 ones_v, zeros1d_v, cntbuf_v, semA, semB):
    with_cnt = cnt_hbm is not None
    c = lax.axis_index("c")
    s = lax.axis_index("s")
    r = c * NTILE + s
    rbase = r * RCAP

    pltpu.sync_copy(cnts_hbm.at[pl.ds(r * 16, 16)], cntbuf_v)
    nb = cntbuf_v[...][0]

    zv = jnp.zeros((16,), jnp.float32)

    def zrow(i, carry):
        for l in range(C // 16):
            rowsA[i, pl.ds(l * 16, 16)] = zv
        return carry
    lax.fori_loop(0, K, zrow, 0)

    def z1d(i, carry):
        zeros1d_v[pl.ds(i * 16, 16)] = zv
        return carry
    lax.fori_loop(0, ROWS_ZERO_TILE // 16, z1d, 0)

    if with_cnt:
        ov = jnp.ones((16,), jnp.float32)
        for j in range(K // 16):
            ones_v[pl.ds(j * 16, 16)] = ov

    z0 = s * ROWS_ZERO_TILE
    for off in range(0, ROWS_ZERO_TILE, K):
        nrows = min(K, ROWS_ZERO_TILE - off)
        pltpu.sync_copy(rowsA.at[pl.ds(0, nrows)],
                        acc_sh.at[pl.ds(z0 + off, nrows)])
    if with_cnt:
        pltpu.sync_copy(zeros1d_v, cnt_sh.at[pl.ds(z0, ROWS_ZERO_TILE)])
    plsc.subcore_barrier()

    def batch_body(b, carry):
        eoff = rbase + b * KB
        pltpu.sync_copy(srcp_hbm.at[pl.ds(eoff, KB)], srcidx_v)
        pltpu.sync_copy(dstp_hbm.at[pl.ds(eoff, KB)], dstraw_v)
        for j in range(KB // 16):
            dstrel_v[j // (K // 16), pl.ds((j % (K // 16)) * 16, 16)] = (
                dstraw_v[pl.ds(j * 16, 16)])

        def start_gather(k, buf, sem):
            return pltpu.async_copy(
                h_hbm.at[srcidx_v.at[pl.ds(k * K, K)]], buf, sem)

        g = start_gather(0, rowsA, semA)
        for k in range(IB):
            cur_buf = rowsA if k % 2 == 0 else rowsB
            nxt_buf = rowsB if k % 2 == 0 else rowsA
            nxt_sem = semB if k % 2 == 0 else semA
            g_next = start_gather(k + 1, nxt_buf, nxt_sem) if k + 1 < IB else None
            g.wait()
            if with_cnt:
                pltpu.sync_copy(ones_v, cnt_sh.at[dstrel_v.at[k]], add=True)
            pltpu.sync_copy(cur_buf, acc_sh.at[dstrel_v.at[k]], add=True)
            g = g_next
        return carry
    lax.fori_loop(0, nb, batch_body, 0)
    plsc.subcore_barrier()

    abase = s * ROWS_OUT_TILE
    obase = c * NH + s * ROWS_OUT_TILE
    for aoff in range(0, ROWS_OUT_TILE, K):
        nrows = min(K, ROWS_OUT_TILE - aoff)
        pltpu.sync_copy(acc_sh.at[pl.ds(abase + aoff, nrows)],
                        rowsA.at[pl.ds(0, nrows)])
        pltpu.sync_copy(rowsA.at[pl.ds(0, nrows)],
                        out_hbm.at[pl.ds(obase + aoff, nrows)])
    if with_cnt:
        pltpu.sync_copy(cnt_sh.at[pl.ds(abase, ROWS_OUT_TILE)],
                        zeros1d_v.at[pl.ds(0, ROWS_OUT_TILE)])
        pltpu.sync_copy(zeros1d_v.at[pl.ds(0, ROWS_OUT_TILE)],
                        cnt_hbm.at[pl.ds(obase, ROWS_OUT_TILE)])


def _sc_body_cnt(h_hbm, srcp_hbm, dstp_hbm, cnts_hbm, out_hbm, cnt_hbm,
                 acc_sh, cnt_sh, rowsA, rowsB, srcidx_v, dstraw_v, dstrel_v,
                 ones_v, zeros1d_v, cntbuf_v, semA, semB):
    _sc_agg_impl(h_hbm, srcp_hbm, dstp_hbm, cnts_hbm, out_hbm, cnt_hbm,
                 acc_sh, cnt_sh, rowsA, rowsB, srcidx_v, dstraw_v, dstrel_v,
                 ones_v, zeros1d_v, cntbuf_v, semA, semB)


def _sc_body_nocnt(h_hbm, srcp_hbm, dstp_hbm, cnts_hbm, out_hbm, acc_sh,
                   rowsA, rowsB, srcidx_v, dstraw_v, dstrel_v,
                   zeros1d_v, cntbuf_v, semA, semB):
    _sc_agg_impl(h_hbm, srcp_hbm, dstp_hbm, cnts_hbm, out_hbm, None,
                 acc_sh, None, rowsA, rowsB, srcidx_v, dstraw_v, dstrel_v,
                 None, zeros1d_v, cntbuf_v, semA, semB)


def _sc_common_scratch():
    return [
        pltpu.VMEM((K, C), jnp.float32),
        pltpu.VMEM((K, C), jnp.float32),
        pltpu.VMEM((KB,), jnp.int32),
        pltpu.VMEM((KB,), jnp.int32),
        pltpu.VMEM((IB, K), jnp.int32),
    ]


@functools.cache
def _get_sc_aggs():
    mesh = plsc.VectorSubcoreMesh(core_axis_name="c", subcore_axis_name="s",
                                  num_cores=NSC, num_subcores=NTILE)
    agg_cnt = pl.kernel(
        _sc_body_cnt,
        out_type=[jax.ShapeDtypeStruct((N, C), jnp.float32),
                  jax.ShapeDtypeStruct((N,), jnp.float32)],
        mesh=mesh,
        scratch_types=[
            pltpu.VMEM_SHARED((NHT, C), jnp.float32),
            pltpu.VMEM_SHARED((NHT,), jnp.float32),
            *_sc_common_scratch(),
            pltpu.VMEM((K,), jnp.float32),
            pltpu.VMEM((ROWS_ZERO_TILE,), jnp.float32),
            pltpu.VMEM((16,), jnp.int32),
            pltpu.SemaphoreType.DMA,
            pltpu.SemaphoreType.DMA,
        ],
    )
    agg_nocnt = pl.kernel(
        _sc_body_nocnt,
        out_type=jax.ShapeDtypeStruct((N, C), jnp.float32),
        mesh=mesh,
        scratch_types=[
            pltpu.VMEM_SHARED((NHT, C), jnp.float32),
            *_sc_common_scratch(),
            pltpu.VMEM((ROWS_ZERO_TILE,), jnp.float32),
            pltpu.VMEM((16,), jnp.int32),
            pltpu.SemaphoreType.DMA,
            pltpu.SemaphoreType.DMA,
        ],
    )
    return agg_cnt, agg_nocnt


_R = 3584
_G = N // _R


def _normw(w):
    n = jnp.sqrt(jnp.sum(w * w, axis=1, keepdims=True))
    return w / (EPS + n * INV_SQRT_C)


def _mm_nt(a, w):
    return lax.dot_general(a, w, (((1,), (1,)), ((), ())),
                           preferred_element_type=jnp.float32)


def _mpconv_body(x_ref, w_ref, o_ref):
    wn = _normw(w_ref[...]) * INV_SQRT_C
    o_ref[...] = _mm_nt(x_ref[...], wn)


def _sage_dense(h, sums, cnt, lw, lb, rw):
    agg = sums * (1.0 / jnp.maximum(cnt, 1.0))
    lwn = _normw(lw) * INV_SQRT_C
    rwn = _normw(rw) * INV_SQRT_C
    out = _mm_nt(agg, lwn) + lb + _mm_nt(h, rwn)
    nrm = jnp.sqrt(jnp.sum(out * out, axis=1, keepdims=True))
    return out / jnp.maximum(nrm, 1e-12)


def _sage_body(h_ref, sums_ref, cnt_ref, lw_ref, lb_ref, rw_ref, o_ref):
    o_ref[...] = _sage_dense(h_ref[...], sums_ref[...], cnt_ref[...],
                             lw_ref[...], lb_ref[...], rw_ref[...])


def _sage_final_body(h_ref, sums_ref, cnt_ref, lw_ref, lb_ref, rw_ref,
                     w3_ref, o_ref):
    hn = _sage_dense(h_ref[...], sums_ref[...], cnt_ref[...], lw_ref[...],
                     lb_ref[...], rw_ref[...])
    w3n = _normw(w3_ref[...]) * INV_SQRT_C
    o_ref[...] = _mm_nt(hn, w3n)


_row_spec = pl.BlockSpec((_R, C), lambda i: (i, 0))
_cnt_spec = pl.BlockSpec((_R, 1), lambda i: (i, 0))
_w_spec = pl.BlockSpec((C, C), lambda i: (0, 0))
_b_spec = pl.BlockSpec((1, C), lambda i: (0, 0))
_out_sds = jax.ShapeDtypeStruct((N, C), jnp.float32)

_mpconv = pl.pallas_call(
    _mpconv_body, grid=(_G,),
    in_specs=[_row_spec, _w_spec],
    out_specs=_row_spec, out_shape=_out_sds)

_sage_tc = pl.pallas_call(
    _sage_body, grid=(_G,),
    in_specs=[_row_spec, _row_spec, _cnt_spec, _w_spec, _b_spec, _w_spec],
    out_specs=_row_spec, out_shape=_out_sds)

_sage_final_tc = pl.pallas_call(
    _sage_final_body, grid=(_G,),
    in_specs=[_row_spec, _row_spec, _cnt_spec, _w_spec, _b_spec, _w_spec,
              _w_spec],
    out_specs=_row_spec, out_shape=_out_sds)


def kernel(x, edge_index, w0, l1_lw, l1_lb, l1_rw, l2_lw, l2_lb, l2_rw, w3):
    nodes = jnp.transpose(x, (0, 2, 3, 1)).reshape(-1, C)
    src = edge_index[0]
    dst = edge_index[1]
    partition = _get_partition()
    agg_cnt, agg_nocnt = _get_sc_aggs()
    srcp, dstp, cnts = partition(src, dst)
    h0 = _mpconv(nodes, w0)
    sums1, cnt1 = agg_cnt(h0, srcp, dstp, cnts)
    cnt1c = cnt1.reshape(N, 1)
    h1 = _sage_tc(h0, sums1, cnt1c, l1_lw, l1_lb.reshape(1, C), l1_rw)
    sums2 = agg_nocnt(h1, srcp, dstp, cnts)
    y = _sage_final_tc(h1, sums2, cnt1c, l2_lw, l2_lb.reshape(1, C), l2_rw, w3)
    return jnp.transpose(y.reshape(B, H, W, C), (0, 3, 1, 2))


# speedup vs baseline: 9.1934x; 1.2656x over previous
import functools

import numpy as np
import jax
import jax.numpy as jnp
from jax import lax
from jax.experimental import pallas as pl
from jax.experimental.pallas import tpu as pltpu
from jax.experimental.pallas import tpu_sc as plsc

B, C, H, W = 2, 128, 112, 112
N = B * H * W
E = 401408
EPS = 1e-4

NSC = 2
NTILE = 16
NW = NSC * NTILE
NH = N // NSC
NTRASH = 256
NHT = NH + NTRASH
K = 64
IB = 8
KB = K * IB
PK = KB
EDGES_PER_TILE = E // NTILE
PART_KB = 896
PART_BATCHES = EDGES_PER_TILE // PART_KB
RCAP = EDGES_PER_TILE
ROWS_OUT_TILE = NH // NTILE
ROWS_ZERO_TILE = NHT // NTILE
INV_SQRT_C = 1.0 / np.sqrt(float(C))


def _part_body(src_hbm, dst_hbm, srcp_hbm, dstp_hbm, cnts_hbm,
               srcraw_v, dstraw_v, pkstg_v, srcstg_v, dststg_v, cntbuf_v,
               semS, semD):
    c = lax.axis_index("c")
    s = lax.axis_index("s")
    r = c * NTILE + s
    base = c * NH
    rbase = r * RCAP
    iota = lax.iota(jnp.int32, 16)

    def flush(p, nf):
        cond = p >= PK

        @pl.when(cond)
        def _():
            @pl.when(nf > 0)
            def _():
                pltpu.make_async_copy(
                    srcstg_v,
                    srcp_hbm.at[pl.ds(rbase + (nf - 1) * PK, PK)],
                    semS).wait()
                pltpu.make_async_copy(
                    dststg_v,
                    dstp_hbm.at[pl.ds(rbase + (nf - 1) * PK, PK)],
                    semD).wait()
            for q in range(PK // 16):
                v = pkstg_v[pl.ds(q * 16, 16)]
                srcstg_v[pl.ds(q * 16, 16)] = v >> 14
                dststg_v[pl.ds(q * 16, 16)] = v & 16383
            pltpu.async_copy(srcstg_v,
                             srcp_hbm.at[pl.ds(rbase + nf * PK, PK)], semS)
            pltpu.async_copy(dststg_v,
                             dstp_hbm.at[pl.ds(rbase + nf * PK, PK)], semD)
            for q in range(8):
                v = pkstg_v[pl.ds(PK + q * 16, 16)]
                pkstg_v[pl.ds(q * 16, 16)] = v
        p2 = jnp.where(cond, p - PK, p)
        nf2 = jnp.where(cond, nf + 1, nf)
        return p2, nf2

    def batch_body(b, carry):
        p, f = carry
        eoff = s * EDGES_PER_TILE + b * PART_KB
        pltpu.sync_copy(src_hbm.at[pl.ds(eoff, PART_KB)], srcraw_v)
        pltpu.sync_copy(dst_hbm.at[pl.ds(eoff, PART_KB)], dstraw_v)
        for j in range(PART_KB // 16):
            sr = srcraw_v[pl.ds(j * 16, 16)]
            dr = dstraw_v[pl.ds(j * 16, 16)]
            rel = dr - base
            ok = (rel >= 0) & (rel < NH)
            key = jnp.where(ok, 0, 1).astype(jnp.int32)
            packed = (sr << 14) | (rel & 16383)
            _, sval = plsc.sort_key_val(key, packed)
            cnt16 = plsc.all_reduce_population_count(ok)[0]
            plsc.store_scatter(pkstg_v, [p + iota], sval,
                               mask=iota < cnt16)
            p = p + cnt16
            if j % 8 == 7:
                p, f = flush(p, f)
        return p, f

    p, f = lax.fori_loop(0, PART_BATCHES, batch_body,
                         (jnp.int32(0), jnp.int32(0)))

    need16 = (16 - (p % 16)) % 16
    mask = iota < need16
    plsc.store_scatter(pkstg_v, [p + iota], (iota << 14) | (NH + iota),
                       mask=mask)
    p = p + need16
    p, f = flush(p, f)

    total = p + f * PK
    n_push = jnp.where(total == 0, PK // 16, ((PK - (p % PK)) % PK) // 16)

    def push_body(j, carry):
        p, f = carry
        plsc.store_scatter(pkstg_v, [p + iota],
                           (iota << 14) | (NH + (j % 16) * 16 + iota))
        p = p + 16
        p, f = flush(p, f)
        return p, f
    p, f = lax.fori_loop(0, n_push, push_body, (p, f))

    @pl.when(f > 0)
    def _():
        pltpu.make_async_copy(
            srcstg_v, srcp_hbm.at[pl.ds(rbase + (f - 1) * PK, PK)],
            semS).wait()
        pltpu.make_async_copy(
            dststg_v, dstp_hbm.at[pl.ds(rbase + (f - 1) * PK, PK)],
            semD).wait()
    cntbuf_v[...] = jnp.zeros((16,), jnp.int32) + f
    pltpu.sync_copy(cntbuf_v, cnts_hbm.at[pl.ds(r * 16, 16)])


@functools.cache
def _get_partition():
    mesh = plsc.VectorSubcoreMesh(core_axis_name="c", subcore_axis_name="s",
                                  num_cores=NSC, num_subcores=NTILE)
    return pl.kernel(
        _part_body,
        out_type=[jax.ShapeDtypeStruct((NW * RCAP,), jnp.int32),
                  jax.ShapeDtypeStruct((NW * RCAP,), jnp.int32),
                  jax.ShapeDtypeStruct((NW * 16,), jnp.int32)],
        mesh=mesh,
        compiler_params=pltpu.CompilerParams(needs_layout_passes=False),
        scratch_types=[
            pltpu.VMEM((PART_KB,), jnp.int32),
            pltpu.VMEM((PART_KB,), jnp.int32),
            pltpu.VMEM((PK + 128,), jnp.int32),
            pltpu.VMEM((PK,), jnp.int32),
            pltpu.VMEM((PK,), jnp.int32),
            pltpu.VMEM((16,), jnp.int32),
            pltpu.SemaphoreType.DMA,
            pltpu.SemaphoreType.DMA,
        ],
    )


def _sc_agg_impl(h_hbm, srcp_hbm, dstp_hbm, cnts_hbm, out_hbm, cnt_hbm,
                 acc_sh, cnt_sh, rowsA, rowsB, srcidx_v, dstraw_v, dstrel_v,
                 ones_v, zeros1d_v, cntbuf_v, semA, semB):
    with_cnt = cnt_hbm is not None
    c = lax.axis_index("c")
    s = lax.axis_index("s")
    r = c * NTILE + s
    rbase = r * RCAP

    pltpu.sync_copy(cnts_hbm.at[pl.ds(r * 16, 16)], cntbuf_v)
    nb = cntbuf_v[...][0]

    zv = jnp.zeros((16,), jnp.float32)

    def zrow(i, carry):
        for l in range(C // 16):
            rowsA[i, pl.ds(l * 16, 16)] = zv
        return carry
    lax.fori_loop(0, K, zrow, 0)

    def z1d(i, carry):
        zeros1d_v[pl.ds(i * 16, 16)] = zv
        return carry
    lax.fori_loop(0, ROWS_ZERO_TILE // 16, z1d, 0)

    if with_cnt:
        ov = jnp.ones((16,), jnp.float32)
        for j in range(K // 16):
            ones_v[pl.ds(j * 16, 16)] = ov

    z0 = s * ROWS_ZERO_TILE
    for off in range(0, ROWS_ZERO_TILE, K):
        nrows = min(K, ROWS_ZERO_TILE - off)
        pltpu.sync_copy(rowsA.at[pl.ds(0, nrows)],
                        acc_sh.at[pl.ds(z0 + off, nrows)])
    if with_cnt:
        pltpu.sync_copy(zeros1d_v, cnt_sh.at[pl.ds(z0, ROWS_ZERO_TILE)])
    plsc.subcore_barrier()

    def batch_body(b, carry):
        eoff = rbase + b * KB
        pltpu.sync_copy(srcp_hbm.at[pl.ds(eoff, KB)], srcidx_v)
        pltpu.sync_copy(dstp_hbm.at[pl.ds(eoff, KB)], dstraw_v)
        for j in range(KB // 16):
            dstrel_v[j // (K // 16), pl.ds((j % (K // 16)) * 16, 16)] = (
                dstraw_v[pl.ds(j * 16, 16)])

        def start_gather(k, buf, sem):
            return pltpu.async_copy(
                h_hbm.at[srcidx_v.at[pl.ds(k * K, K)]], buf, sem)

        g = start_gather(0, rowsA, semA)
        for k in range(IB):
            cur_buf = rowsA if k % 2 == 0 else rowsB
            nxt_buf = rowsB if k % 2 == 0 else rowsA
            nxt_sem = semB if k % 2 == 0 else semA
            g_next = start_gather(k + 1, nxt_buf, nxt_sem) if k + 1 < IB else None
            g.wait()
            if with_cnt:
                pltpu.sync_copy(ones_v, cnt_sh.at[dstrel_v.at[k]], add=True)
            pltpu.sync_copy(cur_buf, acc_sh.at[dstrel_v.at[k]], add=True)
            g = g_next
        return carry
    lax.fori_loop(0, nb, batch_body, 0)
    plsc.subcore_barrier()

    abase = s * ROWS_OUT_TILE
    obase = c * NH + s * ROWS_OUT_TILE
    for aoff in range(0, ROWS_OUT_TILE, K):
        nrows = min(K, ROWS_OUT_TILE - aoff)
        pltpu.sync_copy(acc_sh.at[pl.ds(abase + aoff, nrows)],
                        rowsA.at[pl.ds(0, nrows)])
        pltpu.sync_copy(rowsA.at[pl.ds(0, nrows)],
                        out_hbm.at[pl.ds(obase + aoff, nrows)])
    if with_cnt:
        pltpu.sync_copy(cnt_sh.at[pl.ds(abase, ROWS_OUT_TILE)],
                        zeros1d_v.at[pl.ds(0, ROWS_OUT_TILE)])
        pltpu.sync_copy(zeros1d_v.at[pl.ds(0, ROWS_OUT_TILE)],
                        cnt_hbm.at[pl.ds(obase, ROWS_OUT_TILE)])


def _sc_body_cnt(h_hbm, srcp_hbm, dstp_hbm, cnts_hbm, out_hbm, cnt_hbm,
                 acc_sh, cnt_sh, rowsA, rowsB, srcidx_v, dstraw_v, dstrel_v,
                 ones_v, zeros1d_v, cntbuf_v, semA, semB):
    _sc_agg_impl(h_hbm, srcp_hbm, dstp_hbm, cnts_hbm, out_hbm, cnt_hbm,
                 acc_sh, cnt_sh, rowsA, rowsB, srcidx_v, dstraw_v, dstrel_v,
                 ones_v, zeros1d_v, cntbuf_v, semA, semB)


def _sc_body_nocnt(h_hbm, srcp_hbm, dstp_hbm, cnts_hbm, out_hbm, acc_sh,
                   rowsA, rowsB, srcidx_v, dstraw_v, dstrel_v,
                   zeros1d_v, cntbuf_v, semA, semB):
    _sc_agg_impl(h_hbm, srcp_hbm, dstp_hbm, cnts_hbm, out_hbm, None,
                 acc_sh, None, rowsA, rowsB, srcidx_v, dstraw_v, dstrel_v,
                 None, zeros1d_v, cntbuf_v, semA, semB)


def _sc_common_scratch():
    return [
        pltpu.VMEM((K, C), jnp.float32),
        pltpu.VMEM((K, C), jnp.float32),
        pltpu.VMEM((KB,), jnp.int32),
        pltpu.VMEM((KB,), jnp.int32),
        pltpu.VMEM((IB, K), jnp.int32),
    ]


@functools.cache
def _get_sc_aggs():
    mesh = plsc.VectorSubcoreMesh(core_axis_name="c", subcore_axis_name="s",
                                  num_cores=NSC, num_subcores=NTILE)
    agg_cnt = pl.kernel(
        _sc_body_cnt,
        out_type=[jax.ShapeDtypeStruct((N, C), jnp.float32),
                  jax.ShapeDtypeStruct((N,), jnp.float32)],
        mesh=mesh,
        scratch_types=[
            pltpu.VMEM_SHARED((NHT, C), jnp.float32),
            pltpu.VMEM_SHARED((NHT,), jnp.float32),
            *_sc_common_scratch(),
            pltpu.VMEM((K,), jnp.float32),
            pltpu.VMEM((ROWS_ZERO_TILE,), jnp.float32),
            pltpu.VMEM((16,), jnp.int32),
            pltpu.SemaphoreType.DMA,
            pltpu.SemaphoreType.DMA,
        ],
    )
    agg_nocnt = pl.kernel(
        _sc_body_nocnt,
        out_type=jax.ShapeDtypeStruct((N, C), jnp.float32),
        mesh=mesh,
        scratch_types=[
            pltpu.VMEM_SHARED((NHT, C), jnp.float32),
            *_sc_common_scratch(),
            pltpu.VMEM((ROWS_ZERO_TILE,), jnp.float32),
            pltpu.VMEM((16,), jnp.int32),
            pltpu.SemaphoreType.DMA,
            pltpu.SemaphoreType.DMA,
        ],
    )
    return agg_cnt, agg_nocnt


_R = 3584
_G = N // _R


def _normw(w):
    n = jnp.sqrt(jnp.sum(w * w, axis=1, keepdims=True))
    return w / (EPS + n * INV_SQRT_C)


def _mm_nt(a, w):
    return lax.dot_general(a, w, (((1,), (1,)), ((), ())),
                           preferred_element_type=jnp.float32)


def _mpconv_body(x_ref, w_ref, o_ref):
    wn = _normw(w_ref[...]) * INV_SQRT_C
    o_ref[...] = _mm_nt(x_ref[...], wn)


def _sage_dense(h, sums, cnt, lw, lb, rw):
    agg = sums * (1.0 / jnp.maximum(cnt, 1.0))
    lwn = _normw(lw) * INV_SQRT_C
    rwn = _normw(rw) * INV_SQRT_C
    out = _mm_nt(agg, lwn) + lb + _mm_nt(h, rwn)
    nrm = jnp.sqrt(jnp.sum(out * out, axis=1, keepdims=True))
    return out / jnp.maximum(nrm, 1e-12)


def _sage_body(h_ref, sums_ref, cnt_ref, lw_ref, lb_ref, rw_ref, o_ref):
    o_ref[...] = _sage_dense(h_ref[...], sums_ref[...], cnt_ref[...],
                             lw_ref[...], lb_ref[...], rw_ref[...])


def _sage_final_body(h_ref, sums_ref, cnt_ref, lw_ref, lb_ref, rw_ref,
                     w3_ref, o_ref):
    hn = _sage_dense(h_ref[...], sums_ref[...], cnt_ref[...], lw_ref[...],
                     lb_ref[...], rw_ref[...])
    w3n = _normw(w3_ref[...]) * INV_SQRT_C
    o_ref[...] = _mm_nt(hn, w3n)


_row_spec = pl.BlockSpec((_R, C), lambda i: (i, 0))
_cnt_spec = pl.BlockSpec((_R, 1), lambda i: (i, 0))
_w_spec = pl.BlockSpec((C, C), lambda i: (0, 0))
_b_spec = pl.BlockSpec((1, C), lambda i: (0, 0))
_out_sds = jax.ShapeDtypeStruct((N, C), jnp.float32)

_mpconv = pl.pallas_call(
    _mpconv_body, grid=(_G,),
    in_specs=[_row_spec, _w_spec],
    out_specs=_row_spec, out_shape=_out_sds)

_sage_tc = pl.pallas_call(
    _sage_body, grid=(_G,),
    in_specs=[_row_spec, _row_spec, _cnt_spec, _w_spec, _b_spec, _w_spec],
    out_specs=_row_spec, out_shape=_out_sds)

_sage_final_tc = pl.pallas_call(
    _sage_final_body, grid=(_G,),
    in_specs=[_row_spec, _row_spec, _cnt_spec, _w_spec, _b_spec, _w_spec,
              _w_spec],
    out_specs=_row_spec, out_shape=_out_sds)


def kernel(x, edge_index, w0, l1_lw, l1_lb, l1_rw, l2_lw, l2_lb, l2_rw, w3):
    nodes = jnp.transpose(x, (0, 2, 3, 1)).reshape(-1, C)
    src = edge_index[0]
    dst = edge_index[1]
    partition = _get_partition()
    agg_cnt, agg_nocnt = _get_sc_aggs()
    srcp, dstp, cnts = partition(src, dst)
    h0 = _mpconv(nodes, w0)
    sums1, cnt1 = agg_cnt(h0, srcp, dstp, cnts)
    cnt1c = cnt1.reshape(N, 1)
    h1 = _sage_tc(h0, sums1, cnt1c, l1_lw, l1_lb.reshape(1, C), l1_rw)
    sums2 = agg_nocnt(h1, srcp, dstp, cnts)
    y = _sage_final_tc(h1, sums2, cnt1c, l2_lw, l2_lb.reshape(1, C), l2_rw, w3)
    return jnp.transpose(y.reshape(B, H, W, C), (0, 3, 1, 2))


# speedup vs baseline: 9.1949x; 1.0002x over previous
import functools

import numpy as np
import jax
import jax.numpy as jnp
from jax import lax
from jax.experimental import pallas as pl
from jax.experimental.pallas import tpu as pltpu
from jax.experimental.pallas import tpu_sc as plsc

B, C, H, W = 2, 128, 112, 112
N = B * H * W
E = 401408
EPS = 1e-4

NSC = 2
NTILE = 16
NW = NSC * NTILE
NH = N // NSC
NTRASH = 256
NHT = NH + NTRASH
K = 64
IB = 8
KB = K * IB
PK = KB
EDGES_PER_TILE = E // NTILE
PART_KB = 896
PART_BATCHES = EDGES_PER_TILE // PART_KB
RCAP = EDGES_PER_TILE
ROWS_OUT_TILE = NH // NTILE
ROWS_ZERO_TILE = NHT // NTILE
INV_SQRT_C = 1.0 / np.sqrt(float(C))


def _part_body(src_hbm, dst_hbm, srcp_hbm, dstp_hbm, cnts_hbm,
               srcraw_v, dstraw_v, pkstg_v, srcstg_v, dststg_v, cntbuf_v,
               semS, semD):
    c = lax.axis_index("c")
    s = lax.axis_index("s")
    r = c * NTILE + s
    base = c * NH
    rbase = r * RCAP
    iota = lax.iota(jnp.int32, 16)

    def flush(p, nf):
        cond = p >= PK

        @pl.when(cond)
        def _():
            @pl.when(nf > 0)
            def _():
                pltpu.make_async_copy(
                    srcstg_v,
                    srcp_hbm.at[pl.ds(rbase + (nf - 1) * PK, PK)],
                    semS).wait()
                pltpu.make_async_copy(
                    dststg_v,
                    dstp_hbm.at[pl.ds(rbase + (nf - 1) * PK, PK)],
                    semD).wait()
            for q in range(PK // 16):
                v = pkstg_v[pl.ds(q * 16, 16)]
                srcstg_v[pl.ds(q * 16, 16)] = v >> 14
                dststg_v[pl.ds(q * 16, 16)] = v & 16383
            pltpu.async_copy(srcstg_v,
                             srcp_hbm.at[pl.ds(rbase + nf * PK, PK)], semS)
            pltpu.async_copy(dststg_v,
                             dstp_hbm.at[pl.ds(rbase + nf * PK, PK)], semD)
            for q in range(8):
                v = pkstg_v[pl.ds(PK + q * 16, 16)]
                pkstg_v[pl.ds(q * 16, 16)] = v
        p2 = jnp.where(cond, p - PK, p)
        nf2 = jnp.where(cond, nf + 1, nf)
        return p2, nf2

    def batch_body(b, carry):
        p, f = carry
        eoff = s * EDGES_PER_TILE + b * PART_KB
        pltpu.sync_copy(src_hbm.at[pl.ds(eoff, PART_KB)], srcraw_v)
        pltpu.sync_copy(dst_hbm.at[pl.ds(eoff, PART_KB)], dstraw_v)
        for j in range(PART_KB // 16):
            sr = srcraw_v[pl.ds(j * 16, 16)]
            dr = dstraw_v[pl.ds(j * 16, 16)]
            rel = dr - base
            ok = (rel >= 0) & (rel < NH)
            key = jnp.where(ok, 0, 1).astype(jnp.int32)
            packed = (sr << 14) | (rel & 16383)
            _, sval = plsc.sort_key_val(key, packed)
            cnt16 = plsc.all_reduce_population_count(ok)[0]
            plsc.store_scatter(pkstg_v, [p + iota], sval,
                               mask=iota < cnt16)
            p = p + cnt16
            if j % 8 == 7:
                p, f = flush(p, f)
        return p, f

    p, f = lax.fori_loop(0, PART_BATCHES, batch_body,
                         (jnp.int32(0), jnp.int32(0)))

    need16 = (16 - (p % 16)) % 16
    mask = iota < need16
    plsc.store_scatter(pkstg_v, [p + iota], (iota << 14) | (NH + iota),
                       mask=mask)
    p = p + need16
    p, f = flush(p, f)

    total = p + f * PK
    n_push = jnp.where(total == 0, PK // 16, ((PK - (p % PK)) % PK) // 16)

    def push_body(j, carry):
        p, f = carry
        plsc.store_scatter(pkstg_v, [p + iota],
                           (iota << 14) | (NH + (j % 16) * 16 + iota))
        p = p + 16
        p, f = flush(p, f)
        return p, f
    p, f = lax.fori_loop(0, n_push, push_body, (p, f))

    @pl.when(f > 0)
    def _():
        pltpu.make_async_copy(
            srcstg_v, srcp_hbm.at[pl.ds(rbase + (f - 1) * PK, PK)],
            semS).wait()
        pltpu.make_async_copy(
            dststg_v, dstp_hbm.at[pl.ds(rbase + (f - 1) * PK, PK)],
            semD).wait()
    cntbuf_v[...] = jnp.zeros((16,), jnp.int32) + f
    pltpu.sync_copy(cntbuf_v, cnts_hbm.at[pl.ds(r * 16, 16)])


@functools.cache
def _get_partition():
    mesh = plsc.VectorSubcoreMesh(core_axis_name="c", subcore_axis_name="s",
                                  num_cores=NSC, num_subcores=NTILE)
    return pl.kernel(
        _part_body,
        out_type=[jax.ShapeDtypeStruct((NW * RCAP,), jnp.int32),
                  jax.ShapeDtypeStruct((NW * RCAP,), jnp.int32),
                  jax.ShapeDtypeStruct((NW * 16,), jnp.int32)],
        mesh=mesh,
        compiler_params=pltpu.CompilerParams(needs_layout_passes=False),
        cost_estimate=pl.CostEstimate(flops=0, transcendentals=0,
                                      bytes_accessed=13_000_000),
        scratch_types=[
            pltpu.VMEM((PART_KB,), jnp.int32),
            pltpu.VMEM((PART_KB,), jnp.int32),
            pltpu.VMEM((PK + 128,), jnp.int32),
            pltpu.VMEM((PK,), jnp.int32),
            pltpu.VMEM((PK,), jnp.int32),
            pltpu.VMEM((16,), jnp.int32),
            pltpu.SemaphoreType.DMA,
            pltpu.SemaphoreType.DMA,
        ],
    )


def _sc_agg_impl(h_hbm, srcp_hbm, dstp_hbm, cnts_hbm, out_hbm, cnt_hbm,
                 acc_sh, cnt_sh, rowsA, rowsB, srcidx_v, dstraw_v, dstrel_v,
                 ones_v, zeros1d_v, cntbuf_v, semA, semB):
    with_cnt = cnt_hbm is not None
    c = lax.axis_index("c")
    s = lax.axis_index("s")
    r = c * NTILE + s
    rbase = r * RCAP

    pltpu.sync_copy(cnts_hbm.at[pl.ds(r * 16, 16)], cntbuf_v)
    nb = cntbuf_v[...][0]

    zv = jnp.zeros((16,), jnp.float32)

    def zrow(i, carry):
        for l in range(C // 16):
            rowsA[i, pl.ds(l * 16, 16)] = zv
        return carry
    lax.fori_loop(0, K, zrow, 0)

    def z1d(i, carry):
        zeros1d_v[pl.ds(i * 16, 16)] = zv
        return carry
    lax.fori_loop(0, ROWS_ZERO_TILE // 16, z1d, 0)

    if with_cnt:
        ov = jnp.ones((16,), jnp.float32)
        for j in range(K // 16):
            ones_v[pl.ds(j * 16, 16)] = ov

    z0 = s * ROWS_ZERO_TILE
    for off in range(0, ROWS_ZERO_TILE, K):
        nrows = min(K, ROWS_ZERO_TILE - off)
        pltpu.sync_copy(rowsA.at[pl.ds(0, nrows)],
                        acc_sh.at[pl.ds(z0 + off, nrows)])
    if with_cnt:
        pltpu.sync_copy(zeros1d_v, cnt_sh.at[pl.ds(z0, ROWS_ZERO_TILE)])
    plsc.subcore_barrier()

    def batch_body(b, carry):
        eoff = rbase + b * KB
        pltpu.sync_copy(srcp_hbm.at[pl.ds(eoff, KB)], srcidx_v)
        pltpu.sync_copy(dstp_hbm.at[pl.ds(eoff, KB)], dstraw_v)
        for j in range(KB // 16):
            dstrel_v[j // (K // 16), pl.ds((j % (K // 16)) * 16, 16)] = (
                dstraw_v[pl.ds(j * 16, 16)])

        def start_gather(k, buf, sem):
            return pltpu.async_copy(
                h_hbm.at[srcidx_v.at[pl.ds(k * K, K)]], buf, sem)

        g = start_gather(0, rowsA, semA)
        for k in range(IB):
            cur_buf = rowsA if k % 2 == 0 else rowsB
            nxt_buf = rowsB if k % 2 == 0 else rowsA
            nxt_sem = semB if k % 2 == 0 else semA
            g_next = start_gather(k + 1, nxt_buf, nxt_sem) if k + 1 < IB else None
            g.wait()
            if with_cnt:
                pltpu.sync_copy(ones_v, cnt_sh.at[dstrel_v.at[k]], add=True)
            pltpu.sync_copy(cur_buf, acc_sh.at[dstrel_v.at[k]], add=True)
            g = g_next
        return carry
    lax.fori_loop(0, nb, batch_body, 0)
    plsc.subcore_barrier()

    abase = s * ROWS_OUT_TILE
    obase = c * NH + s * ROWS_OUT_TILE
    for aoff in range(0, ROWS_OUT_TILE, K):
        nrows = min(K, ROWS_OUT_TILE - aoff)
        pltpu.sync_copy(acc_sh.at[pl.ds(abase + aoff, nrows)],
                        rowsA.at[pl.ds(0, nrows)])
        pltpu.sync_copy(rowsA.at[pl.ds(0, nrows)],
                        out_hbm.at[pl.ds(obase + aoff, nrows)])
    if with_cnt:
        pltpu.sync_copy(cnt_sh.at[pl.ds(abase, ROWS_OUT_TILE)],
                        zeros1d_v.at[pl.ds(0, ROWS_OUT_TILE)])
        pltpu.sync_copy(zeros1d_v.at[pl.ds(0, ROWS_OUT_TILE)],
                        cnt_hbm.at[pl.ds(obase, ROWS_OUT_TILE)])


def _sc_body_cnt(h_hbm, srcp_hbm, dstp_hbm, cnts_hbm, out_hbm, cnt_hbm,
                 acc_sh, cnt_sh, rowsA, rowsB, srcidx_v, dstraw_v, dstrel_v,
                 ones_v, zeros1d_v, cntbuf_v, semA, semB):
    _sc_agg_impl(h_hbm, srcp_hbm, dstp_hbm, cnts_hbm, out_hbm, cnt_hbm,
                 acc_sh, cnt_sh, rowsA, rowsB, srcidx_v, dstraw_v, dstrel_v,
                 ones_v, zeros1d_v, cntbuf_v, semA, semB)


def _sc_body_nocnt(h_hbm, srcp_hbm, dstp_hbm, cnts_hbm, out_hbm, acc_sh,
                   rowsA, rowsB, srcidx_v, dstraw_v, dstrel_v,
                   zeros1d_v, cntbuf_v, semA, semB):
    _sc_agg_impl(h_hbm, srcp_hbm, dstp_hbm, cnts_hbm, out_hbm, None,
                 acc_sh, None, rowsA, rowsB, srcidx_v, dstraw_v, dstrel_v,
                 None, zeros1d_v, cntbuf_v, semA, semB)


def _sc_common_scratch():
    return [
        pltpu.VMEM((K, C), jnp.float32),
        pltpu.VMEM((K, C), jnp.float32),
        pltpu.VMEM((KB,), jnp.int32),
        pltpu.VMEM((KB,), jnp.int32),
        pltpu.VMEM((IB, K), jnp.int32),
    ]


@functools.cache
def _get_sc_aggs():
    mesh = plsc.VectorSubcoreMesh(core_axis_name="c", subcore_axis_name="s",
                                  num_cores=NSC, num_subcores=NTILE)
    agg_cnt = pl.kernel(
        _sc_body_cnt,
        out_type=[jax.ShapeDtypeStruct((N, C), jnp.float32),
                  jax.ShapeDtypeStruct((N,), jnp.float32)],
        mesh=mesh,
        scratch_types=[
            pltpu.VMEM_SHARED((NHT, C), jnp.float32),
            pltpu.VMEM_SHARED((NHT,), jnp.float32),
            *_sc_common_scratch(),
            pltpu.VMEM((K,), jnp.float32),
            pltpu.VMEM((ROWS_ZERO_TILE,), jnp.float32),
            pltpu.VMEM((16,), jnp.int32),
            pltpu.SemaphoreType.DMA,
            pltpu.SemaphoreType.DMA,
        ],
    )
    agg_nocnt = pl.kernel(
        _sc_body_nocnt,
        out_type=jax.ShapeDtypeStruct((N, C), jnp.float32),
        mesh=mesh,
        scratch_types=[
            pltpu.VMEM_SHARED((NHT, C), jnp.float32),
            *_sc_common_scratch(),
            pltpu.VMEM((ROWS_ZERO_TILE,), jnp.float32),
            pltpu.VMEM((16,), jnp.int32),
            pltpu.SemaphoreType.DMA,
            pltpu.SemaphoreType.DMA,
        ],
    )
    return agg_cnt, agg_nocnt


_R = 3584
_G = N // _R


def _normw(w):
    n = jnp.sqrt(jnp.sum(w * w, axis=1, keepdims=True))
    return w / (EPS + n * INV_SQRT_C)


def _mm_nt(a, w):
    return lax.dot_general(a, w, (((1,), (1,)), ((), ())),
                           preferred_element_type=jnp.float32)


def _mpconv_body(x_ref, w_ref, o_ref):
    wn = _normw(w_ref[...]) * INV_SQRT_C
    o_ref[...] = _mm_nt(x_ref[...], wn)


def _sage_dense(h, sums, cnt, lw, lb, rw):
    agg = sums * (1.0 / jnp.maximum(cnt, 1.0))
    lwn = _normw(lw) * INV_SQRT_C
    rwn = _normw(rw) * INV_SQRT_C
    out = _mm_nt(agg, lwn) + lb + _mm_nt(h, rwn)
    nrm = jnp.sqrt(jnp.sum(out * out, axis=1, keepdims=True))
    return out / jnp.maximum(nrm, 1e-12)


def _sage_body(h_ref, sums_ref, cnt_ref, lw_ref, lb_ref, rw_ref, o_ref):
    o_ref[...] = _sage_dense(h_ref[...], sums_ref[...], cnt_ref[...],
                             lw_ref[...], lb_ref[...], rw_ref[...])


def _sage_final_body(h_ref, sums_ref, cnt_ref, lw_ref, lb_ref, rw_ref,
                     w3_ref, o_ref):
    hn = _sage_dense(h_ref[...], sums_ref[...], cnt_ref[...], lw_ref[...],
                     lb_ref[...], rw_ref[...])
    w3n = _normw(w3_ref[...]) * INV_SQRT_C
    o_ref[...] = _mm_nt(hn, w3n)


_row_spec = pl.BlockSpec((_R, C), lambda i: (i, 0))
_cnt_spec = pl.BlockSpec((_R, 1), lambda i: (i, 0))
_w_spec = pl.BlockSpec((C, C), lambda i: (0, 0))
_b_spec = pl.BlockSpec((1, C), lambda i: (0, 0))
_out_sds = jax.ShapeDtypeStruct((N, C), jnp.float32)

_mpconv = pl.pallas_call(
    _mpconv_body, grid=(_G,),
    in_specs=[_row_spec, _w_spec],
    out_specs=_row_spec, out_shape=_out_sds)

_sage_tc = pl.pallas_call(
    _sage_body, grid=(_G,),
    in_specs=[_row_spec, _row_spec, _cnt_spec, _w_spec, _b_spec, _w_spec],
    out_specs=_row_spec, out_shape=_out_sds)

_sage_final_tc = pl.pallas_call(
    _sage_final_body, grid=(_G,),
    in_specs=[_row_spec, _row_spec, _cnt_spec, _w_spec, _b_spec, _w_spec,
              _w_spec],
    out_specs=_row_spec, out_shape=_out_sds)


def kernel(x, edge_index, w0, l1_lw, l1_lb, l1_rw, l2_lw, l2_lb, l2_rw, w3):
    nodes = jnp.transpose(x, (0, 2, 3, 1)).reshape(-1, C)
    src = edge_index[0]
    dst = edge_index[1]
    partition = _get_partition()
    agg_cnt, agg_nocnt = _get_sc_aggs()
    srcp, dstp, cnts = partition(src, dst)
    h0 = _mpconv(nodes, w0)
    sums1, cnt1 = agg_cnt(h0, srcp, dstp, cnts)
    cnt1c = cnt1.reshape(N, 1)
    h1 = _sage_tc(h0, sums1, cnt1c, l1_lw, l1_lb.reshape(1, C), l1_rw)
    sums2 = agg_nocnt(h1, srcp, dstp, cnts)
    y = _sage_final_tc(h1, sums2, cnt1c, l2_lw, l2_lb.reshape(1, C), l2_rw, w3)
    return jnp.transpose(y.reshape(B, H, W, C), (0, 3, 1, 2))
